# baseline (speedup 1.0000x reference)
import functools

import jax
import jax.numpy as jnp
from jax import lax
from jax.experimental import pallas as pl
from jax.experimental.pallas import tpu as pltpu
from jax.experimental.pallas import tpu_sc as plsc

N = 10000
E = 320000
NC = 2
NS = 16
CHE = 128
EPT = 20480
NCHT = EPT // CHE
NPH = 8
E_PAD = NS * EPT
N_PAD = 10112
ZPT = N_PAD // NS
TPT = N // NS


def _make_agg(DH):
  D = 2 * DH
  NB = 4
  mesh = plsc.VectorSubcoreMesh(core_axis_name="c", subcore_axis_name="s")

  @functools.partial(
      pl.kernel,
      out_type=jax.ShapeDtypeStruct((N_PAD, D), jnp.float32),
      mesh=mesh,
      compiler_params=pltpu.CompilerParams(use_tc_tiling_on_sc=False),
      scratch_types=[
          pltpu.VMEM((2, NCHT // NPH, CHE), jnp.int32),
          pltpu.VMEM((2, NCHT // NPH, CHE), jnp.int32),
          pltpu.VMEM((NB, CHE, DH), jnp.float32),
          pltpu.VMEM_SHARED((N_PAD, DH), jnp.float32),
          pltpu.VMEM_SHARED((N_PAD, DH), jnp.float32),
          pltpu.SemaphoreType.DMA,
      ] + [pltpu.SemaphoreType.DMA] * (2 * NB + 2),
  )
  def agg(g_hbm, ep_hbm, out_hbm,
          srcv, dstv, rows, tab, acc, semt, *sems):
    sg = sems[:NB]
    ss = sems[NB:2 * NB]
    si = sems[2 * NB:]
    c = lax.axis_index("c")
    s = lax.axis_index("s")
    nch_p = NCHT // NPH

    tcopy = pltpu.async_copy(
        g_hbm.at[pl.ds(s * TPT, TPT), pl.ds(c * DH, DH)],
        tab.at[pl.ds(s * TPT, TPT)], semt)
    z16 = jnp.zeros((16,), jnp.float32)

    def zz(j, carry):
      for l in range(DH // 16):
        rows[0, j, pl.ds(l * 16, 16)] = z16
      return carry

    lax.fori_loop(0, CHE, zz, 0)
    for r in range(ZPT // CHE):
      pltpu.sync_copy(rows.at[0], acc.at[pl.ds(s * ZPT + r * CHE, CHE)])
    rem = ZPT - (ZPT // CHE) * CHE
    pltpu.sync_copy(rows.at[0, pl.ds(0, rem)],
                    acc.at[pl.ds(s * ZPT + (ZPT // CHE) * CHE, rem)])
    tcopy.wait()
    plsc.subcore_barrier()

    def idx_copies(p, q):
      pltpu.async_copy(ep_hbm.at[0, s, pl.ds(p * nch_p, nch_p)],
                       srcv.at[q], si[q])
      pltpu.async_copy(ep_hbm.at[1, s, pl.ds(p * nch_p, nch_p)],
                       dstv.at[q], si[q])

    def issue_gather(q, k, b):
      pltpu.async_copy(tab.at[srcv.at[q, k]], rows.at[b], sg[b])

    def wait_gather(q, k, b):
      pltpu.make_async_copy(tab.at[srcv.at[q, k]], rows.at[b], sg[b]).wait()

    def issue_scatter(q, k, b):
      pltpu.async_copy(rows.at[b], acc.at[dstv.at[q, k]], ss[b], add=True)

    def wait_scatter(b):
      pltpu.make_async_copy(rows.at[b], acc.at[dstv.at[0, 0]], ss[b]).wait()

    def visit(q, v, b, swait, gissue):
      if swait:
        wait_scatter((b + 2) % NB)
      if gissue:
        issue_gather(q, v + 2, (b + 2) % NB)
      wait_gather(q, v, b)
      issue_scatter(q, v, b)

    idx_copies(0, 0)
    for p in range(NPH):
      q = p % 2
      if p + 1 < NPH:
        idx_copies(p + 1, 1 - q)
      pltpu.make_async_copy(ep_hbm.at[0, s, pl.ds(p * nch_p, nch_p)],
                            srcv.at[q], si[q]).wait()
      pltpu.make_async_copy(ep_hbm.at[1, s, pl.ds(p * nch_p, nch_p)],
                            dstv.at[q], si[q]).wait()
      issue_gather(q, 0, 0)
      issue_gather(q, 1, 1)
      visit(q, 0, 0, False, True)
      visit(q, 1, 1, False, True)
      visit(q, 2, 2, True, True)
      visit(q, 3, 3, True, True)

      def step(i, carry):
        v0 = 4 + i * 4
        for j in range(NB):
          visit(q, v0 + j, j, True, True)
        return carry

      lax.fori_loop(0, (nch_p - 8) // 4, step, 0)
      visit(q, nch_p - 4, 0, True, True)
      visit(q, nch_p - 3, 1, True, True)
      visit(q, nch_p - 2, 2, True, False)
      visit(q, nch_p - 1, 3, True, False)
      wait_scatter(2)
      wait_scatter(3)

    plsc.subcore_barrier()
    pltpu.sync_copy(acc.at[pl.ds(s * ZPT, ZPT)],
                    out_hbm.at[pl.ds(s * ZPT, ZPT), pl.ds(c * DH, DH)])

  return agg


_agg64 = _make_agg(64)
_agg32 = _make_agg(32)


def _mm_body(x_ref, w_ref, o_ref):
  o_ref[...] = jnp.dot(x_ref[...], w_ref[...],
                       preferred_element_type=jnp.float32)


def _combine_mm_body(g_ref, p_ref, b_ref, w_ref, o_ref):
  h = jnp.maximum(g_ref[...] + p_ref[...] + b_ref[...], 0.0)
  o_ref[...] = jnp.dot(h, w_ref[...], preferred_element_type=jnp.float32)


def _combine_body(g_ref, p_ref, b_ref, o_ref):
  o_ref[...] = g_ref[...] + p_ref[...] + b_ref[...]


_BM = 1000


def _mm(x, w):
  n, d = x.shape
  h = w.shape[1]
  return pl.pallas_call(
      _mm_body,
      grid=(n // _BM,),
      in_specs=[
          pl.BlockSpec((_BM, d), lambda i: (i, 0)),
          pl.BlockSpec((d, h), lambda i: (0, 0)),
      ],
      out_specs=pl.BlockSpec((_BM, h), lambda i: (i, 0)),
      out_shape=jax.ShapeDtypeStruct((n, h), jnp.float32),
  )(x, w)


def _combine_mm(g, p, b, w):
  n, d = g.shape
  h = w.shape[1]
  return pl.pallas_call(
      _combine_mm_body,
      grid=(n // _BM,),
      in_specs=[
          pl.BlockSpec((_BM, d), lambda i: (i, 0)),
          pl.BlockSpec((_BM, d), lambda i: (i, 0)),
          pl.BlockSpec((1, d), lambda i: (0, 0)),
          pl.BlockSpec((d, h), lambda i: (0, 0)),
      ],
      out_specs=pl.BlockSpec((_BM, h), lambda i: (i, 0)),
      out_shape=jax.ShapeDtypeStruct((n, h), jnp.float32),
  )(g, p, b.reshape(1, d), w)


def _combine(g, p, b):
  n, d = g.shape
  return pl.pallas_call(
      _combine_body,
      grid=(n // _BM,),
      in_specs=[
          pl.BlockSpec((_BM, d), lambda i: (i, 0)),
          pl.BlockSpec((_BM, d), lambda i: (i, 0)),
          pl.BlockSpec((1, d), lambda i: (0, 0)),
      ],
      out_specs=pl.BlockSpec((_BM, d), lambda i: (i, 0)),
      out_shape=jax.ShapeDtypeStruct((n, d), jnp.float32),
  )(g, p, b.reshape(1, d))


def kernel(x, edge_index, W1, b1, W2, b2, W3, b3):
  ep = jnp.pad(edge_index, ((0, 0), (0, E_PAD - E)),
               constant_values=N).reshape(2, NS, NCHT, CHE)

  g1 = _mm(x, W1)
  p1 = _agg64(g1, ep)
  g2 = _combine_mm(g1, p1, b1, W2)
  p2 = _agg64(g2, ep)
  g3 = _combine_mm(g2, p2, b2, W3)
  p3 = _agg32(g3, ep)
  return _combine(g3, p3, b3)


# speedup vs baseline: 1.0608x; 1.0608x over previous
import functools

import jax
import jax.numpy as jnp
from jax import lax
from jax.experimental import pallas as pl
from jax.experimental.pallas import tpu as pltpu
from jax.experimental.pallas import tpu_sc as plsc

N = 10000
E = 320000
NC = 2
NS = 16
CHE = 128
EPT = 20480
NCHT = EPT // CHE
NPH = 4
E_PAD = NS * EPT
N_PAD = 10112
ZPT = N_PAD // NS
TPT = N // NS


def _make_agg(DH):
  D = 2 * DH
  NB = 4
  mesh = plsc.VectorSubcoreMesh(core_axis_name="c", subcore_axis_name="s")

  @functools.partial(
      pl.kernel,
      out_type=jax.ShapeDtypeStruct((N_PAD, D), jnp.float32),
      mesh=mesh,
      compiler_params=pltpu.CompilerParams(use_tc_tiling_on_sc=False),
      scratch_types=[
          pltpu.VMEM((NCHT // NPH, CHE), jnp.int32),
          pltpu.VMEM((NCHT // NPH, CHE), jnp.int32),
          pltpu.VMEM((NB, CHE, DH), jnp.float32),
          pltpu.VMEM_SHARED((N_PAD, DH), jnp.float32),
          pltpu.VMEM_SHARED((N_PAD, DH), jnp.float32),
          pltpu.SemaphoreType.DMA,
      ] + [pltpu.SemaphoreType.DMA] * (2 * NB),
  )
  def agg(g_hbm, ep_hbm, out_hbm,
          srcv, dstv, rows, tab, acc, semt, *sems):
    sg = sems[:NB]
    ss = sems[NB:]
    c = lax.axis_index("c")
    s = lax.axis_index("s")
    nch_p = NCHT // NPH

    tcopy = pltpu.async_copy(
        g_hbm.at[pl.ds(s * TPT, TPT), pl.ds(c * DH, DH)],
        tab.at[pl.ds(s * TPT, TPT)], semt)
    z16 = jnp.zeros((16,), jnp.float32)

    def zz(j, carry):
      for l in range(DH // 16):
        rows[0, j, pl.ds(l * 16, 16)] = z16
      return carry

    lax.fori_loop(0, CHE, zz, 0)
    for r in range(ZPT // CHE):
      pltpu.sync_copy(rows.at[0], acc.at[pl.ds(s * ZPT + r * CHE, CHE)])
    rem = ZPT - (ZPT // CHE) * CHE
    pltpu.sync_copy(rows.at[0, pl.ds(0, rem)],
                    acc.at[pl.ds(s * ZPT + (ZPT // CHE) * CHE, rem)])
    tcopy.wait()
    plsc.subcore_barrier()

    def issue_gather(k, b):
      pltpu.async_copy(tab.at[srcv.at[k]], rows.at[b], sg[b])

    def wait_gather(k, b):
      pltpu.make_async_copy(tab.at[srcv.at[k]], rows.at[b], sg[b]).wait()

    def issue_scatter(k, b):
      pltpu.async_copy(rows.at[b], acc.at[dstv.at[k]], ss[b], add=True)

    def wait_scatter(b):
      pltpu.make_async_copy(rows.at[b], acc.at[dstv.at[0]], ss[b]).wait()

    def visit(v, b, swait, gissue):
      if swait:
        wait_scatter((b + 2) % NB)
      if gissue:
        issue_gather(v + 2, (b + 2) % NB)
      wait_gather(v, b)
      issue_scatter(v, b)

    for p in range(NPH):
      pltpu.sync_copy(ep_hbm.at[0, s, pl.ds(p * nch_p, nch_p)], srcv)
      pltpu.sync_copy(ep_hbm.at[1, s, pl.ds(p * nch_p, nch_p)], dstv)
      issue_gather(0, 0)
      issue_gather(1, 1)
      visit(0, 0, False, True)
      visit(1, 1, False, True)
      visit(2, 2, True, True)
      visit(3, 3, True, True)

      def step(i, carry):
        v0 = 4 + i * 4
        for j in range(NB):
          visit(v0 + j, j, True, True)
        return carry

      lax.fori_loop(0, (nch_p - 8) // 4, step, 0)
      visit(nch_p - 4, 0, True, True)
      visit(nch_p - 3, 1, True, True)
      visit(nch_p - 2, 2, True, False)
      visit(nch_p - 1, 3, True, False)
      wait_scatter(2)
      wait_scatter(3)

    plsc.subcore_barrier()
    pltpu.sync_copy(acc.at[pl.ds(s * ZPT, ZPT)],
                    out_hbm.at[pl.ds(s * ZPT, ZPT), pl.ds(c * DH, DH)])

  return agg


_agg64 = _make_agg(64)
_agg32 = _make_agg(32)


def _mm_body(x_ref, w_ref, o_ref):
  o_ref[...] = jnp.dot(x_ref[...], w_ref[...],
                       preferred_element_type=jnp.float32)


def _combine_mm_body(g_ref, p_ref, b_ref, w_ref, o_ref):
  h = jnp.maximum(g_ref[...] + p_ref[...] + b_ref[...], 0.0)
  o_ref[...] = jnp.dot(h, w_ref[...], preferred_element_type=jnp.float32)


def _combine_body(g_ref, p_ref, b_ref, o_ref):
  o_ref[...] = g_ref[...] + p_ref[...] + b_ref[...]


_BM = 2000


def _mm(x, w):
  n, d = x.shape
  h = w.shape[1]
  return pl.pallas_call(
      _mm_body,
      grid=(n // _BM,),
      in_specs=[
          pl.BlockSpec((_BM, d), lambda i: (i, 0)),
          pl.BlockSpec((d, h), lambda i: (0, 0)),
      ],
      out_specs=pl.BlockSpec((_BM, h), lambda i: (i, 0)),
      out_shape=jax.ShapeDtypeStruct((n, h), jnp.float32),
  )(x, w)


def _combine_mm(g, p, b, w):
  n, d = g.shape
  h = w.shape[1]
  return pl.pallas_call(
      _combine_mm_body,
      grid=(n // _BM,),
      in_specs=[
          pl.BlockSpec((_BM, d), lambda i: (i, 0)),
          pl.BlockSpec((_BM, d), lambda i: (i, 0)),
          pl.BlockSpec((1, d), lambda i: (0, 0)),
          pl.BlockSpec((d, h), lambda i: (0, 0)),
      ],
      out_specs=pl.BlockSpec((_BM, h), lambda i: (i, 0)),
      out_shape=jax.ShapeDtypeStruct((n, h), jnp.float32),
  )(g, p, b.reshape(1, d), w)


def _combine(g, p, b):
  n, d = g.shape
  return pl.pallas_call(
      _combine_body,
      grid=(n // _BM,),
      in_specs=[
          pl.BlockSpec((_BM, d), lambda i: (i, 0)),
          pl.BlockSpec((_BM, d), lambda i: (i, 0)),
          pl.BlockSpec((1, d), lambda i: (0, 0)),
      ],
      out_specs=pl.BlockSpec((_BM, d), lambda i: (i, 0)),
      out_shape=jax.ShapeDtypeStruct((n, d), jnp.float32),
  )(g, p, b.reshape(1, d))


def kernel(x, edge_index, W1, b1, W2, b2, W3, b3):
  ep = jnp.pad(edge_index, ((0, 0), (0, E_PAD - E)),
               constant_values=N).reshape(2, NS, NCHT, CHE)

  g1 = _mm(x, W1)
  p1 = _agg64(g1, ep)
  g2 = _combine_mm(g1, p1, b1, W2)
  p2 = _agg64(g2, ep)
  g3 = _combine_mm(g2, p2, b2, W3)
  p3 = _agg32(g3, ep)
  return _combine(g3, p3, b3)
